# baseline (speedup 1.0000x reference)
import functools

import jax
import jax.numpy as jnp
from jax import lax
from jax.experimental import pallas as pl
from jax.experimental.pallas import tpu as pltpu
from jax.experimental.pallas import tpu_sc as plsc

_N = 10000
_NPAD = 10240
_E = 320000
_D = 128

_NC = 2
_NS = 16
_NW = _NC * _NS

_SUB = 40
_NSUB = 25
_CHUNK = _SUB * _NSUB
_EPW = _E // _NW
_NIT = _EPW // _CHUNK
_NCHUNKS = _E // _CHUNK

_NBUF = 6
_LEAD = 3
_DC = 128

_RPS = _NPAD // _NS
_ZR = 32

_f32 = jnp.float32
_i32 = jnp.int32
_i16 = jnp.int16

_mesh = plsc.VectorSubcoreMesh(
    core_axis_name="c", subcore_axis_name="s",
    num_cores=_NC, num_subcores=_NS)


def _fill_rows(ref, nrows, ncols, value, dtype=_f32):
    if dtype == _i16:
        vec = jnp.full((2, 16), value, dtype)
        cpr = ncols // 16

        def body(i, _):
            r = (i // cpr) * 2
            c = (i % cpr) * 16
            ref[pl.ds(r, 2), pl.ds(c, 16)] = vec
            return 0

        lax.fori_loop(0, (nrows // 2) * cpr, body, 0)
        return

    vec = jnp.full((16,), value, dtype)
    cpr = ncols // 16

    def body(i, _):
        r = i // cpr
        c = (i % cpr) * 16
        ref[r, pl.ds(c, 16)] = vec
        return 0

    lax.fori_loop(0, nrows * cpr, body, 0)


@functools.partial(
    pl.kernel,
    out_type=jax.ShapeDtypeStruct((_NC, _NPAD, _D), _f32),
    mesh=_mesh,
    scratch_types=(
        pltpu.VMEM((_NSUB, _SUB), _i32),
        pltpu.VMEM((_NSUB, _SUB), _i32),
        tuple(pltpu.VMEM((_SUB, _D), _f32) for _ in range(_NBUF)),
        pltpu.VMEM((_ZR, _D), _f32),
        pltpu.VMEM_SHARED((_NPAD, _D), _f32),
        tuple(pltpu.SemaphoreType.DMA for _ in range(_NBUF)),
        tuple(pltpu.SemaphoreType.DMA for _ in range(_NBUF)),
    ),
)
def _agg(x_hbm, src3, dst3, acc_out, src_v, dst_v, rows, zbuf_v, acc_sh,
         gsems, ssems):
    cid = lax.axis_index("c")
    sid = lax.axis_index("s")
    wid = sid * _NC + cid

    _fill_rows(zbuf_v, _ZR, _D, 0.0)
    row0 = sid * _RPS
    for k in range(_RPS // _ZR):
        pltpu.sync_copy(zbuf_v, acc_sh.at[pl.ds(row0 + k * _ZR, _ZR)])
    plsc.subcore_barrier()

    def gather(j):
        b = j % _NBUF
        return pltpu.async_copy(x_hbm.at[src_v.at[j]], rows[b], gsems[b])

    def scatter(j):
        b = j % _NBUF
        return pltpu.async_copy(rows[b], acc_sh.at[dst_v.at[j]], ssems[b],
                                add=True)

    def step(i, _):
        g = wid * _NIT + i
        pltpu.sync_copy(src3.at[g], src_v)
        pltpu.sync_copy(dst3.at[g], dst_v)
        gds = {j: gather(j) for j in range(_LEAD)}
        sds = {}
        for j in range(_NSUB):
            nj = j + _LEAD
            if nj < _NSUB:
                if nj - _NBUF >= 0:
                    sds.pop(nj - _NBUF).wait()
                gds[nj] = gather(nj)
            gds.pop(j).wait()
            sds[j] = scatter(j)
        for j in sorted(sds):
            sds[j].wait()
        return 0

    lax.fori_loop(0, _NIT, step, 0)
    plsc.subcore_barrier()

    pltpu.sync_copy(
        acc_sh.at[pl.ds(row0, _RPS)],
        acc_out.at[cid, pl.ds(row0, _RPS)],
    )


@functools.partial(
    pl.kernel,
    out_type=jax.ShapeDtypeStruct((_NC, _NPAD, _DC), _f32),
    mesh=_mesh,
    scratch_types=(
        pltpu.VMEM((_NSUB, _SUB), _i32),
        pltpu.VMEM((_SUB, _DC), _f32),
        pltpu.VMEM((_ZR, _DC), _f32),
        pltpu.VMEM_SHARED((_NPAD, _DC), _f32),
    ),
)
def _cnt(dst3, cnt_out, dst_v, ones_v, zcnt_v, cnt_sh):
    cid = lax.axis_index("c")
    sid = lax.axis_index("s")
    wid = sid * _NC + cid

    _fill_rows(zcnt_v, _ZR, _DC, 0.0)
    _fill_rows(ones_v, _SUB, _DC, 1.0)
    row0 = sid * _RPS
    for k in range(_RPS // _ZR):
        pltpu.sync_copy(zcnt_v, cnt_sh.at[pl.ds(row0 + k * _ZR, _ZR)])
    plsc.subcore_barrier()

    def step(i, _):
        g = wid * _NIT + i
        pltpu.sync_copy(dst3.at[g], dst_v)
        for j in range(_NSUB):
            pltpu.sync_copy(ones_v, cnt_sh.at[dst_v.at[j]], add=True)
        return 0

    lax.fori_loop(0, _NIT, step, 0)
    plsc.subcore_barrier()

    pltpu.sync_copy(
        cnt_sh.at[pl.ds(row0, _RPS)],
        cnt_out.at[cid, pl.ds(row0, _RPS)],
    )


_BM = 400


def _tc_layer(acc, cnt, xin, wl, wc, b, relu):
    def body(a_ref, c_ref, x_ref, wl_ref, wc_ref, b_ref, o_ref):
        cvals = (c_ref[0, :, 0:1] + c_ref[1, :, 0:1]).astype(_f32)
        cnt_col = jnp.maximum(cvals, 1.0)
        agg = (a_ref[0] + a_ref[1]) / cnt_col
        y = jnp.dot(agg, wl_ref[...], preferred_element_type=_f32)
        y = y + jnp.dot(x_ref[...], wc_ref[...], preferred_element_type=_f32)
        y = y + b_ref[...]
        if relu:
            y = jnp.maximum(y, 0.0)
        o_ref[...] = y

    row = pl.BlockSpec((_BM, _D), lambda i: (i, 0))
    full = pl.BlockSpec((_D, _D), lambda i: (0, 0))
    bias = pl.BlockSpec((1, _D), lambda i: (0, 0))
    return pl.pallas_call(
        body,
        grid=(_N // _BM,),
        in_specs=[
            pl.BlockSpec((_NC, _BM, _D), lambda i: (0, i, 0)),
            pl.BlockSpec((_NC, _BM, _DC), lambda i: (0, i, 0)),
            row, full, full, bias,
        ],
        out_specs=row,
        out_shape=jax.ShapeDtypeStruct((_N, _D), _f32),
    )(acc, cnt, xin, wl, wc, b)


@jax.jit
def kernel(x, edge_index, Wl1, bl1, Wr1, Wlin1, blin1, Wl2, bl2, Wr2, Wlin2,
           blin2):
    src3 = edge_index[0].reshape(_NCHUNKS, _NSUB, _SUB)
    dst3 = edge_index[1].reshape(_NCHUNKS, _NSUB, _SUB)

    cnt = _cnt(dst3)
    acc1 = _agg(x, src3, dst3)
    h = _tc_layer(acc1, cnt, x,
                  Wl1, Wr1 + Wlin1, (bl1 + blin1).reshape(1, _D), relu=True)

    acc2 = _agg(h, src3, dst3)
    out = _tc_layer(acc2, cnt, h,
                    Wl2, Wr2 + Wlin2, (bl2 + blin2).reshape(1, _D),
                    relu=False)
    return out


# speedup vs baseline: 1.0856x; 1.0856x over previous
import functools

import jax
import jax.numpy as jnp
from jax import lax
from jax.experimental import pallas as pl
from jax.experimental.pallas import tpu as pltpu
from jax.experimental.pallas import tpu_sc as plsc

_N = 10000
_NPAD = 10240
_E = 320000
_D = 128

_NC = 2
_NS = 16
_NW = _NC * _NS

_SUB = 80
_NSUB = 25
_CHUNK = _SUB * _NSUB
_EPW = _E // _NW
_NIT = _EPW // _CHUNK
_NCHUNKS = _E // _CHUNK

_NBUF = 3
_LEAD = 1
_DC = 128

_RPS = _NPAD // _NS
_ZR = 32

_f32 = jnp.float32
_i32 = jnp.int32
_i16 = jnp.int16

_mesh = plsc.VectorSubcoreMesh(
    core_axis_name="c", subcore_axis_name="s",
    num_cores=_NC, num_subcores=_NS)


def _fill_rows(ref, nrows, ncols, value, dtype=_f32):
    if dtype == _i16:
        vec = jnp.full((2, 16), value, dtype)
        cpr = ncols // 16

        def body(i, _):
            r = (i // cpr) * 2
            c = (i % cpr) * 16
            ref[pl.ds(r, 2), pl.ds(c, 16)] = vec
            return 0

        lax.fori_loop(0, (nrows // 2) * cpr, body, 0)
        return

    vec = jnp.full((16,), value, dtype)
    cpr = ncols // 16

    def body(i, _):
        r = i // cpr
        c = (i % cpr) * 16
        ref[r, pl.ds(c, 16)] = vec
        return 0

    lax.fori_loop(0, nrows * cpr, body, 0)


def _make_agg(with_cnt):
    out_type = [jax.ShapeDtypeStruct((_NC, _NPAD, _D), _f32)]
    scratch = [
        pltpu.VMEM((_NSUB, _SUB), _i32),
        pltpu.VMEM((_NSUB, _SUB), _i32),
        tuple(pltpu.VMEM((_SUB, _D), _f32) for _ in range(_NBUF)),
        pltpu.VMEM((_ZR, _D), _f32),
        pltpu.VMEM_SHARED((_NPAD, _D), _f32),
        tuple(pltpu.SemaphoreType.DMA for _ in range(_NBUF)),
        tuple(pltpu.SemaphoreType.DMA for _ in range(_NBUF)),
    ]
    if with_cnt:
        out_type.append(jax.ShapeDtypeStruct((_NC, _NPAD, _D), _f32))

    @functools.partial(
        pl.kernel,
        out_type=tuple(out_type),
        mesh=_mesh,
        scratch_types=tuple(scratch),
    )
    def agg(x_hbm, src3, dst3, *rest):
        if with_cnt:
            (acc_out, cnt_out, src_v, dst_v, rows, zbuf_v, acc_sh,
             gsems, ssems) = rest
        else:
            (acc_out, src_v, dst_v, rows, zbuf_v, acc_sh,
             gsems, ssems) = rest

        cid = lax.axis_index("c")
        sid = lax.axis_index("s")
        wid = sid * _NC + cid

        _fill_rows(zbuf_v, _ZR, _D, 0.0)
        row0 = sid * _RPS
        for k in range(_RPS // _ZR):
            pltpu.sync_copy(zbuf_v, acc_sh.at[pl.ds(row0 + k * _ZR, _ZR)])
        plsc.subcore_barrier()

        def gather(j):
            b = j % _NBUF
            return pltpu.async_copy(x_hbm.at[src_v.at[j]], rows[b], gsems[b])

        def scatter(j):
            b = j % _NBUF
            return pltpu.async_copy(rows[b], acc_sh.at[dst_v.at[j]], ssems[b],
                                    add=True)

        def step(i, _):
            g = wid * _NIT + i
            pltpu.sync_copy(src3.at[g], src_v)
            pltpu.sync_copy(dst3.at[g], dst_v)
            gds = {j: gather(j) for j in range(_LEAD)}
            sds = {}
            for j in range(_NSUB):
                nj = j + _LEAD
                if nj < _NSUB:
                    if nj - _NBUF >= 0:
                        sds.pop(nj - _NBUF).wait()
                    gds[nj] = gather(nj)
                gds.pop(j).wait()
                sds[j] = scatter(j)
            for j in sorted(sds):
                sds[j].wait()
            return 0

        lax.fori_loop(0, _NIT, step, 0)
        plsc.subcore_barrier()

        pltpu.sync_copy(
            acc_sh.at[pl.ds(row0, _RPS)],
            acc_out.at[cid, pl.ds(row0, _RPS)],
        )

        if with_cnt:
            ones_v = rows[0]
            _fill_rows(ones_v, _SUB, _D, 1.0)
            for k in range(_RPS // _ZR):
                pltpu.sync_copy(zbuf_v, acc_sh.at[pl.ds(row0 + k * _ZR, _ZR)])
            plsc.subcore_barrier()

            def cstep(i, _):
                g = wid * _NIT + i
                pltpu.sync_copy(dst3.at[g], dst_v)
                for j in range(_NSUB):
                    pltpu.sync_copy(ones_v, acc_sh.at[dst_v.at[j]], add=True)
                return 0

            lax.fori_loop(0, _NIT, cstep, 0)
            plsc.subcore_barrier()

            pltpu.sync_copy(
                acc_sh.at[pl.ds(row0, _RPS)],
                cnt_out.at[cid, pl.ds(row0, _RPS)],
            )

    return agg


_agg_cnt = _make_agg(True)
_agg = _make_agg(False)


_BM = 400


def _tc_layer(acc, cnt, xin, wl, wc, b, relu):
    def body(a_ref, c_ref, x_ref, wl_ref, wc_ref, b_ref, o_ref):
        cvals = (c_ref[0, :, 0:1] + c_ref[1, :, 0:1]).astype(_f32)
        cnt_col = jnp.maximum(cvals, 1.0)
        agg = (a_ref[0] + a_ref[1]) / cnt_col
        y = jnp.dot(agg, wl_ref[...], preferred_element_type=_f32)
        y = y + jnp.dot(x_ref[...], wc_ref[...], preferred_element_type=_f32)
        y = y + b_ref[...]
        if relu:
            y = jnp.maximum(y, 0.0)
        o_ref[...] = y

    row = pl.BlockSpec((_BM, _D), lambda i: (i, 0))
    full = pl.BlockSpec((_D, _D), lambda i: (0, 0))
    bias = pl.BlockSpec((1, _D), lambda i: (0, 0))
    return pl.pallas_call(
        body,
        grid=(_N // _BM,),
        in_specs=[
            pl.BlockSpec((_NC, _BM, _D), lambda i: (0, i, 0)),
            pl.BlockSpec((_NC, _BM, _DC), lambda i: (0, i, 0)),
            row, full, full, bias,
        ],
        out_specs=row,
        out_shape=jax.ShapeDtypeStruct((_N, _D), _f32),
    )(acc, cnt, xin, wl, wc, b)


@jax.jit
def kernel(x, edge_index, Wl1, bl1, Wr1, Wlin1, blin1, Wl2, bl2, Wr2, Wlin2,
           blin2):
    src3 = edge_index[0].reshape(_NCHUNKS, _NSUB, _SUB)
    dst3 = edge_index[1].reshape(_NCHUNKS, _NSUB, _SUB)

    acc1, cnt = _agg_cnt(x, src3, dst3)
    h = _tc_layer(acc1, cnt, x,
                  Wl1, Wr1 + Wlin1, (bl1 + blin1).reshape(1, _D), relu=True)

    (acc2,) = _agg(h, src3, dst3)
    out = _tc_layer(acc2, cnt, h,
                    Wl2, Wr2 + Wlin2, (bl2 + blin2).reshape(1, _D),
                    relu=False)
    return out
